# baseline (speedup 1.0000x reference)
import functools

import jax
import jax.numpy as jnp
from jax import lax
from jax.experimental import pallas as pl
from jax.experimental.pallas import tpu as pltpu
from jax.experimental.pallas import tpu_sc as plsc

N = 256
LS = 2.0
HS = LS / (N - 1)

B = 2097152
NW = 32
PPW = B // NW
CHUNK = 8192
NSTEP = PPW // CHUNK
LANES = 16


def _sc_body(x_hbm, f_hbm, o_hbm, xv, idxv, resv, sem):
    wid = lax.axis_index("s") * 2 + lax.axis_index("c")
    hs = jnp.float32(HS)
    maxv = jnp.float32(N - 1)
    lanes3 = lax.iota(jnp.int32, LANES) * 3

    def quant(v):
        r = (v + 1.0) / hs
        r = jnp.minimum(jnp.maximum(r, 0.0), maxv)
        return r.astype(jnp.int32)

    for k in range(NSTEP):
        base_pt = wid * PPW + k * CHUNK
        pltpu.sync_copy(x_hbm.at[pl.ds(base_pt * 3, CHUNK * 3)], xv)

        def body(j, _):
            ix = lanes3 + j * (3 * LANES)
            i0 = quant(plsc.load_gather(xv, [ix]))
            i1 = quant(plsc.load_gather(xv, [ix + 1]))
            i2 = quant(plsc.load_gather(xv, [ix + 2]))
            lin = i0 * (N * N) + i1 * N + i2
            idxv[pl.ds(j * LANES, LANES)] = lin
            return _

        lax.fori_loop(0, CHUNK // LANES, body, None)

        pltpu.async_copy(f_hbm.at[idxv], resv, sem).wait()
        pltpu.sync_copy(resv, o_hbm.at[pl.ds(base_pt, CHUNK)])


@jax.jit
def kernel(x, f):
    sc_call = pl.kernel(
        _sc_body,
        out_type=jax.ShapeDtypeStruct((B,), jnp.float32),
        mesh=plsc.VectorSubcoreMesh(core_axis_name="c", subcore_axis_name="s"),
        compiler_params=pltpu.CompilerParams(needs_layout_passes=False),
        scratch_types=[
            pltpu.VMEM((3 * CHUNK,), jnp.float32),
            pltpu.VMEM((CHUNK,), jnp.int32),
            pltpu.VMEM((CHUNK,), jnp.float32),
            pltpu.SemaphoreType.DMA,
        ],
    )
    return sc_call(x.reshape(-1), f.reshape(-1))


# speedup vs baseline: 1.0076x; 1.0076x over previous
import functools

import jax
import jax.numpy as jnp
from jax import lax
from jax.experimental import pallas as pl
from jax.experimental.pallas import tpu as pltpu
from jax.experimental.pallas import tpu_sc as plsc

N = 256
LS = 2.0
HS = LS / (N - 1)

B = 2097152
NW = 32
PPW = B // NW
CHUNK = 8192
NSTEP = PPW // CHUNK
LANES = 16


def _sc_body(x_hbm, f_hbm, o_hbm, xv, idxv, resv, sem):
    wid = lax.axis_index("s") * 2 + lax.axis_index("c")
    hs = jnp.float32(HS)
    maxv = jnp.float32(N - 1)
    lanes3 = lax.iota(jnp.int32, LANES) * 3

    def quant(v):
        r = (v + 1.0) / hs
        r = jnp.minimum(jnp.maximum(r, 0.0), maxv)
        return r.astype(jnp.int32)

    for k in range(NSTEP):
        base_pt = wid * PPW + k * CHUNK
        pltpu.sync_copy(x_hbm.at[pl.ds(base_pt * 3, CHUNK * 3)], xv)

        @plsc.parallel_loop(0, CHUNK // LANES, unroll=8)
        def body(j):
            ix = lanes3 + j * (3 * LANES)
            i0 = quant(plsc.load_gather(xv, [ix]))
            i1 = quant(plsc.load_gather(xv, [ix + 1]))
            i2 = quant(plsc.load_gather(xv, [ix + 2]))
            lin = i0 * (N * N) + i1 * N + i2
            idxv[pl.ds(j * LANES, LANES)] = lin

        pltpu.async_copy(f_hbm.at[idxv], resv, sem).wait()
        pltpu.sync_copy(resv, o_hbm.at[pl.ds(base_pt, CHUNK)])


@jax.jit
def kernel(x, f):
    sc_call = pl.kernel(
        _sc_body,
        out_type=jax.ShapeDtypeStruct((B,), jnp.float32),
        mesh=plsc.VectorSubcoreMesh(core_axis_name="c", subcore_axis_name="s"),
        compiler_params=pltpu.CompilerParams(needs_layout_passes=False),
        scratch_types=[
            pltpu.VMEM((3 * CHUNK,), jnp.float32),
            pltpu.VMEM((CHUNK,), jnp.int32),
            pltpu.VMEM((CHUNK,), jnp.float32),
            pltpu.SemaphoreType.DMA,
        ],
    )
    return sc_call(x.reshape(-1), f.reshape(-1))


# speedup vs baseline: 4.2454x; 4.2134x over previous
import functools

import jax
import jax.numpy as jnp
from jax import lax
from jax.experimental import pallas as pl
from jax.experimental.pallas import tpu as pltpu
from jax.experimental.pallas import tpu_sc as plsc

N = 256
LS = 2.0
HS = LS / (N - 1)

B = 2097152
NW = 32
PPW = B // NW
CHUNK = 8192
NSTEP = PPW // CHUNK
LANES = 16


def _sc_body(x_hbm, f_hbm, o_hbm, xv0, xv1, xv2, idxv, resv, sem):
    wid = lax.axis_index("s") * 2 + lax.axis_index("c")
    hs = jnp.float32(HS)
    maxv = jnp.float32(N - 1)

    def quant(v):
        r = (v + 1.0) / hs
        r = jnp.minimum(jnp.maximum(r, 0.0), maxv)
        return r.astype(jnp.int32)

    for k in range(NSTEP):
        base_pt = wid * PPW + k * CHUNK
        pltpu.sync_copy(x_hbm.at[pl.ds(base_pt, CHUNK)], xv0)
        pltpu.sync_copy(x_hbm.at[pl.ds(B + base_pt, CHUNK)], xv1)
        pltpu.sync_copy(x_hbm.at[pl.ds(2 * B + base_pt, CHUNK)], xv2)

        @plsc.parallel_loop(0, CHUNK // LANES, unroll=8)
        def body(j):
            s = pl.ds(j * LANES, LANES)
            i0 = quant(xv0[s])
            i1 = quant(xv1[s])
            i2 = quant(xv2[s])
            idxv[s] = i0 * (N * N) + i1 * N + i2

        pltpu.async_copy(f_hbm.at[idxv], resv, sem).wait()
        pltpu.sync_copy(resv, o_hbm.at[pl.ds(base_pt, CHUNK)])


@jax.jit
def kernel(x, f):
    sc_call = pl.kernel(
        _sc_body,
        out_type=jax.ShapeDtypeStruct((B,), jnp.float32),
        mesh=plsc.VectorSubcoreMesh(core_axis_name="c", subcore_axis_name="s"),
        compiler_params=pltpu.CompilerParams(needs_layout_passes=False),
        scratch_types=[
            pltpu.VMEM((CHUNK,), jnp.float32),
            pltpu.VMEM((CHUNK,), jnp.float32),
            pltpu.VMEM((CHUNK,), jnp.float32),
            pltpu.VMEM((CHUNK,), jnp.int32),
            pltpu.VMEM((CHUNK,), jnp.float32),
            pltpu.SemaphoreType.DMA,
        ],
    )
    one = lax.optimization_barrier(jnp.float32(1.0))
    x_planes = (x * one).T.reshape(3 * B)
    f_lin = (f * one).reshape(N * N * N)
    return sc_call(x_planes, f_lin)


# speedup vs baseline: 8.6665x; 2.0414x over previous
import functools

import jax
import jax.numpy as jnp
from jax import lax
from jax.experimental import pallas as pl
from jax.experimental.pallas import tpu as pltpu
from jax.experimental.pallas import tpu_sc as plsc

N = 256
LS = 2.0
HS = LS / (N - 1)

B = 2097152
NW = 32
PPW = B // NW
CHUNK = 8192
NSTEP = PPW // CHUNK
LANES = 16


def _sc_body(x0_hbm, x1_hbm, x2_hbm, f_hbm, o_hbm, xv0, xv1, xv2, idxv, resv, sem):
    wid = lax.axis_index("s") * 2 + lax.axis_index("c")
    hs = jnp.float32(HS)
    maxv = jnp.float32(N - 1)

    def quant(v):
        r = (v + 1.0) / hs
        r = jnp.minimum(jnp.maximum(r, 0.0), maxv)
        return r.astype(jnp.int32)

    for k in range(NSTEP):
        base_pt = wid * PPW + k * CHUNK
        pltpu.sync_copy(x0_hbm.at[pl.ds(base_pt, CHUNK)], xv0)
        pltpu.sync_copy(x1_hbm.at[pl.ds(base_pt, CHUNK)], xv1)
        pltpu.sync_copy(x2_hbm.at[pl.ds(base_pt, CHUNK)], xv2)

        @plsc.parallel_loop(0, CHUNK // LANES, unroll=8)
        def body(j):
            s = pl.ds(j * LANES, LANES)
            i0 = quant(xv0[s])
            i1 = quant(xv1[s])
            i2 = quant(xv2[s])
            idxv[s] = i0 * (N * N) + i1 * N + i2

        pltpu.async_copy(f_hbm.at[idxv], resv, sem).wait()
        pltpu.sync_copy(resv, o_hbm.at[pl.ds(base_pt, CHUNK)])


@jax.jit
def kernel(x, f):
    sc_call = pl.kernel(
        _sc_body,
        out_type=jax.ShapeDtypeStruct((B,), jnp.float32),
        mesh=plsc.VectorSubcoreMesh(core_axis_name="c", subcore_axis_name="s"),
        compiler_params=pltpu.CompilerParams(needs_layout_passes=False),
        scratch_types=[
            pltpu.VMEM((CHUNK,), jnp.float32),
            pltpu.VMEM((CHUNK,), jnp.float32),
            pltpu.VMEM((CHUNK,), jnp.float32),
            pltpu.VMEM((CHUNK,), jnp.int32),
            pltpu.VMEM((CHUNK,), jnp.float32),
            pltpu.SemaphoreType.DMA,
        ],
    )
    one = lax.optimization_barrier(jnp.float32(1.0))
    x0 = x[:, 0] * one
    x1 = x[:, 1] * one
    x2 = x[:, 2] * one
    f_lin = f.reshape(N * N * N) * one
    return sc_call(x0, x1, x2, f_lin)


# speedup vs baseline: 9.4012x; 1.0848x over previous
import functools

import jax
import jax.numpy as jnp
from jax import lax
from jax.experimental import pallas as pl
from jax.experimental.pallas import tpu as pltpu
from jax.experimental.pallas import tpu_sc as plsc

N = 256
LS = 2.0
HS = LS / (N - 1)

B = 2097152
NW = 32
PPW = B // NW
CHUNK = 8192
NSTEP = PPW // CHUNK
LANES = 16


def _sc_body(
    x0_hbm, x1_hbm, x2_hbm, f_hbm, o_hbm,
    xa0, xa1, xa2, xb0, xb1, xb2, idxa, idxb, resa, resb,
    sxa, sxb, sga, sgb,
):
    wid = lax.axis_index("s") * 2 + lax.axis_index("c")
    hs = jnp.float32(HS)
    maxv = jnp.float32(N - 1)
    xv = ((xa0, xa1, xa2), (xb0, xb1, xb2))
    idxv = (idxa, idxb)
    resv = (resa, resb)
    sx = (sxa, sxb)
    sg = (sga, sgb)

    def quant(v):
        r = (v + 1.0) / hs
        r = jnp.minimum(jnp.maximum(r, 0.0), maxv)
        return r.astype(jnp.int32)

    def start_x(k):
        b = k % 2
        base_pt = wid * PPW + k * CHUNK
        return [
            pltpu.async_copy(xr.at[pl.ds(base_pt, CHUNK)], xv[b][c], sx[b])
            for c, xr in enumerate((x0_hbm, x1_hbm, x2_hbm))
        ]

    hx = {0: start_x(0)}
    hg = {}
    for k in range(NSTEP):
        b = k % 2
        for h in hx[k]:
            h.wait()
        if k + 1 < NSTEP:
            hx[k + 1] = start_x(k + 1)

        @plsc.parallel_loop(0, CHUNK // LANES, unroll=8)
        def body(j):
            s = pl.ds(j * LANES, LANES)
            i0 = quant(xv[b][0][s])
            i1 = quant(xv[b][1][s])
            i2 = quant(xv[b][2][s])
            idxv[b][s] = i0 * (N * N) + i1 * N + i2

        if k >= 1:
            hg[k - 1].wait()
            base_prev = wid * PPW + (k - 1) * CHUNK
            pltpu.sync_copy(resv[1 - b], o_hbm.at[pl.ds(base_prev, CHUNK)])
        hg[k] = pltpu.async_copy(f_hbm.at[idxv[b]], resv[b], sg[b])

    hg[NSTEP - 1].wait()
    base_last = wid * PPW + (NSTEP - 1) * CHUNK
    pltpu.sync_copy(resv[(NSTEP - 1) % 2], o_hbm.at[pl.ds(base_last, CHUNK)])


@jax.jit
def kernel(x, f):
    sc_call = pl.kernel(
        _sc_body,
        out_type=jax.ShapeDtypeStruct((B,), jnp.float32),
        mesh=plsc.VectorSubcoreMesh(core_axis_name="c", subcore_axis_name="s"),
        compiler_params=pltpu.CompilerParams(needs_layout_passes=False),
        scratch_types=[
            pltpu.VMEM((CHUNK,), jnp.float32),
            pltpu.VMEM((CHUNK,), jnp.float32),
            pltpu.VMEM((CHUNK,), jnp.float32),
            pltpu.VMEM((CHUNK,), jnp.float32),
            pltpu.VMEM((CHUNK,), jnp.float32),
            pltpu.VMEM((CHUNK,), jnp.float32),
            pltpu.VMEM((CHUNK,), jnp.int32),
            pltpu.VMEM((CHUNK,), jnp.int32),
            pltpu.VMEM((CHUNK,), jnp.float32),
            pltpu.VMEM((CHUNK,), jnp.float32),
            pltpu.SemaphoreType.DMA,
            pltpu.SemaphoreType.DMA,
            pltpu.SemaphoreType.DMA,
            pltpu.SemaphoreType.DMA,
        ],
    )
    one = lax.optimization_barrier(jnp.float32(1.0))
    x0 = x[:, 0] * one
    x1 = x[:, 1] * one
    x2 = x[:, 2] * one
    f_lin = f.reshape(N * N * N) * one
    return sc_call(x0, x1, x2, f_lin)
